# baseline (device time: 267067 ns/iter reference)
import numpy as np

import jax
import jax.numpy as jnp
from jax import lax
from jax.experimental import pallas as pl
from jax.experimental.pallas import tpu as pltpu

N_DEV = 8
DH = 64


def _rope_tables(B, Sq, H_loc):
    inv = 1.0 / (10000.0 ** (np.arange(0, DH, 2) / DH))
    pos = np.arange(Sq)[:, None] * inv[None, :]
    cos = np.repeat(np.cos(pos), 2, axis=-1).astype(np.float32)
    sin = np.repeat(np.sin(pos), 2, axis=-1).astype(np.float32)
    cos_bc = np.tile(cos, (B, H_loc))
    sin_bc = np.tile(sin, (B, H_loc))
    R = np.zeros((DH, DH), np.float32)
    for k in range(DH // 2):
        R[2 * k, 2 * k + 1] = 1.0
        R[2 * k + 1, 2 * k] = -1.0
    R_full = np.kron(np.eye(H_loc, dtype=np.float32), R)
    return cos_bc, sin_bc, R_full


def kernel(x, Wq, Wk, Wv, Wo):
    B, Sq, D = x.shape
    d_loc = Wq.shape[1]
    H_loc = d_loc // DH
    M = B * Sq

    cos_bc, sin_bc, R_full = _rope_tables(B, Sq, H_loc)
    x2 = x.reshape(M, D)

    def body(x_ref, wq_ref, wk_ref, wv_ref, wo_ref, cos_ref, sin_ref, rot_ref,
             out_ref, qkv_buf, o_buf, ctx_ref,
             qkv_send, qkv_recv, o_send, o_recv):
        my = lax.axis_index("i")
        left = (my + N_DEV - 1) % N_DEV
        right = (my + 1) % N_DEV

        barrier = pltpu.get_barrier_semaphore()
        pl.semaphore_signal(barrier, inc=1, device_id=(left,),
                            device_id_type=pl.DeviceIdType.MESH)
        pl.semaphore_signal(barrier, inc=1, device_id=(right,),
                            device_id_type=pl.DeviceIdType.MESH)
        pl.semaphore_wait(barrier, 2)

        qkv_buf[0, :, 0:d_loc] = wq_ref[...]
        qkv_buf[0, :, d_loc:2 * d_loc] = wk_ref[...]
        qkv_buf[0, :, 2 * d_loc:3 * d_loc] = wv_ref[...]
        o_buf[0] = wo_ref[...]

        def compute(s):
            qkv = jnp.dot(x_ref[...], qkv_buf[s],
                          preferred_element_type=jnp.float32)
            q = qkv[:, 0:d_loc]
            k = qkv[:, d_loc:2 * d_loc]
            v = qkv[:, 2 * d_loc:3 * d_loc]
            cos = cos_ref[...]
            sin = sin_ref[...]
            rot = rot_ref[...]
            q = q * cos + jnp.dot(q, rot, preferred_element_type=jnp.float32) * sin
            k = k * cos + jnp.dot(k, rot, preferred_element_type=jnp.float32) * sin
            for b in range(B):
                for hh in range(H_loc):
                    qs = q[b * Sq:(b + 1) * Sq, hh * DH:(hh + 1) * DH]
                    ks = k[b * Sq:(b + 1) * Sq, hh * DH:(hh + 1) * DH]
                    vs = v[b * Sq:(b + 1) * Sq, hh * DH:(hh + 1) * DH]
                    sc = lax.dot_general(
                        qs, ks, (((1,), (1,)), ((), ())),
                        preferred_element_type=jnp.float32) * 0.125
                    m = jnp.max(sc, axis=1, keepdims=True)
                    w = jnp.exp(sc - m)
                    w = w / jnp.sum(w, axis=1, keepdims=True)
                    ctx_ref[b * Sq:(b + 1) * Sq, hh * DH:(hh + 1) * DH] = (
                        jnp.dot(w, vs, preferred_element_type=jnp.float32))
            return jnp.dot(ctx_ref[...], o_buf[s],
                           preferred_element_type=jnp.float32)

        for h in range(N_DEV - 1):
            qkv_rdma = pltpu.make_async_remote_copy(
                src_ref=qkv_buf.at[h], dst_ref=qkv_buf.at[h + 1],
                send_sem=qkv_send.at[h], recv_sem=qkv_recv.at[h + 1],
                device_id=(right,), device_id_type=pl.DeviceIdType.MESH)
            o_rdma = pltpu.make_async_remote_copy(
                src_ref=o_buf.at[h], dst_ref=o_buf.at[h + 1],
                send_sem=o_send.at[h], recv_sem=o_recv.at[h + 1],
                device_id=(right,), device_id_type=pl.DeviceIdType.MESH)
            qkv_rdma.start()
            o_rdma.start()
            c = compute(h)
            if h == 0:
                out_ref[...] = c
            else:
                out_ref[...] = out_ref[...] + c
            qkv_rdma.wait()
            o_rdma.wait()
        out_ref[...] = out_ref[...] + compute(N_DEV - 1)

    out2 = pl.pallas_call(
        body,
        out_shape=jax.ShapeDtypeStruct((M, D), jnp.float32),
        in_specs=[pl.BlockSpec(memory_space=pltpu.VMEM)] * 8,
        out_specs=pl.BlockSpec(memory_space=pltpu.VMEM),
        scratch_shapes=[
            pltpu.VMEM((N_DEV, D, 3 * d_loc), jnp.float32),
            pltpu.VMEM((N_DEV, d_loc, D), jnp.float32),
            pltpu.VMEM((M, d_loc), jnp.float32),
            pltpu.SemaphoreType.DMA((N_DEV,)),
            pltpu.SemaphoreType.DMA((N_DEV,)),
            pltpu.SemaphoreType.DMA((N_DEV,)),
            pltpu.SemaphoreType.DMA((N_DEV,)),
        ],
        compiler_params=pltpu.CompilerParams(collective_id=0),
    )(x2, Wq, Wk, Wv, Wo,
      jnp.asarray(cos_bc), jnp.asarray(sin_bc), jnp.asarray(R_full))
    return out2.reshape(B, Sq, D)


# device time: 94411 ns/iter; 2.8288x vs baseline; 2.8288x over previous
import numpy as np

import jax
import jax.numpy as jnp
from jax import lax
from jax.experimental import pallas as pl
from jax.experimental.pallas import tpu as pltpu

N_DEV = 8
DH = 64
N_RIGHT = 4
N_LEFT = 3


def _rope_tables(B, Sq, H_loc):
    inv = 1.0 / (10000.0 ** (np.arange(0, DH, 2) / DH))
    pos = np.arange(Sq)[:, None] * inv[None, :]
    cos = np.repeat(np.cos(pos), 2, axis=-1).astype(np.float32)
    sin = np.repeat(np.sin(pos), 2, axis=-1).astype(np.float32)
    cos_bc = np.tile(cos, (B, H_loc))
    sin_bc = np.tile(sin, (B, H_loc))
    R = np.zeros((DH, DH), np.float32)
    for k in range(DH // 2):
        R[2 * k, 2 * k + 1] = 1.0
        R[2 * k + 1, 2 * k] = -1.0
    R_full = np.kron(np.eye(H_loc, dtype=np.float32), R)
    return cos_bc, sin_bc, R_full


def kernel(x, Wq, Wk, Wv, Wo):
    B, Sq, D = x.shape
    d_loc = Wq.shape[1]
    H_loc = d_loc // DH
    M = B * Sq

    cos_bc, sin_bc, R_full = _rope_tables(B, Sq, H_loc)
    x2 = x.reshape(M, D)

    def body(x_ref, wq_ref, wk_ref, wv_ref, wo_ref, cos_ref, sin_ref, rot_ref,
             out_ref, qkv_buf, o_buf, ctx_ref, xb_ref,
             sendR, sendL, recv_sems):
        my = lax.axis_index("i")
        left = (my + N_DEV - 1) % N_DEV
        right = (my + 1) % N_DEV

        barrier = pltpu.get_barrier_semaphore()
        pl.semaphore_signal(barrier, inc=1, device_id=(left,),
                            device_id_type=pl.DeviceIdType.MESH)
        pl.semaphore_signal(barrier, inc=1, device_id=(right,),
                            device_id_type=pl.DeviceIdType.MESH)
        pl.semaphore_wait(barrier, 2)

        xb_ref[...] = x_ref[...].astype(jnp.bfloat16)
        qkv_buf[0, :, 0:d_loc] = wq_ref[...].astype(jnp.bfloat16)
        qkv_buf[0, :, d_loc:2 * d_loc] = wk_ref[...].astype(jnp.bfloat16)
        qkv_buf[0, :, 2 * d_loc:3 * d_loc] = wv_ref[...].astype(jnp.bfloat16)
        o_buf[0] = wo_ref[...].astype(jnp.bfloat16)

        def compute(s):
            qkv = jnp.dot(xb_ref[...], qkv_buf[s],
                          preferred_element_type=jnp.float32)
            q = qkv[:, 0:d_loc]
            k = qkv[:, d_loc:2 * d_loc]
            v = qkv[:, 2 * d_loc:3 * d_loc].astype(jnp.bfloat16)
            cos = cos_ref[...]
            sin = sin_ref[...]
            rot = rot_ref[...]
            q = (q * cos + jnp.dot(q.astype(jnp.bfloat16), rot,
                                   preferred_element_type=jnp.float32) * sin
                 ).astype(jnp.bfloat16)
            k = (k * cos + jnp.dot(k.astype(jnp.bfloat16), rot,
                                   preferred_element_type=jnp.float32) * sin
                 ).astype(jnp.bfloat16)
            for b in range(B):
                for hh in range(H_loc):
                    qs = q[b * Sq:(b + 1) * Sq, hh * DH:(hh + 1) * DH]
                    ks = k[b * Sq:(b + 1) * Sq, hh * DH:(hh + 1) * DH]
                    vs = v[b * Sq:(b + 1) * Sq, hh * DH:(hh + 1) * DH]
                    sc = lax.dot_general(
                        qs, ks, (((1,), (1,)), ((), ())),
                        preferred_element_type=jnp.float32) * 0.125
                    m = jnp.max(sc, axis=1, keepdims=True)
                    w = jnp.exp(sc - m)
                    w = (w / jnp.sum(w, axis=1, keepdims=True)
                         ).astype(jnp.bfloat16)
                    ctx_ref[b * Sq:(b + 1) * Sq, hh * DH:(hh + 1) * DH] = (
                        jnp.dot(w, vs, preferred_element_type=jnp.float32)
                        .astype(jnp.bfloat16))
            return jnp.dot(ctx_ref[...], o_buf[s],
                           preferred_element_type=jnp.float32)

        def make_rdma(src_slot, dst_slot, send_sem, target):
            qkv_rdma = pltpu.make_async_remote_copy(
                src_ref=qkv_buf.at[src_slot], dst_ref=qkv_buf.at[dst_slot],
                send_sem=send_sem.at[2 * dst_slot],
                recv_sem=recv_sems.at[2 * dst_slot],
                device_id=(target,), device_id_type=pl.DeviceIdType.MESH)
            o_rdma = pltpu.make_async_remote_copy(
                src_ref=o_buf.at[src_slot], dst_ref=o_buf.at[dst_slot],
                send_sem=send_sem.at[2 * dst_slot + 1],
                recv_sem=recv_sems.at[2 * dst_slot + 1],
                device_id=(target,), device_id_type=pl.DeviceIdType.MESH)
            return qkv_rdma, o_rdma

        acc_init = [False]

        def accumulate(c):
            if acc_init[0]:
                out_ref[...] = out_ref[...] + c
            else:
                out_ref[...] = c
                acc_init[0] = True

        for r in range(1, N_RIGHT + 1):
            rdmas = list(make_rdma(r - 1, r, sendR, right))
            if r <= N_LEFT:
                src = 0 if r == 1 else 4 + r - 1
                rdmas += list(make_rdma(src, 4 + r, sendL, left))
            for d in rdmas:
                d.start()
            if r == 1:
                accumulate(compute(0))
            else:
                accumulate(compute(r - 1))
                accumulate(compute(4 + r - 1))
            for d in rdmas:
                d.wait()
        accumulate(compute(N_RIGHT))

    out2 = pl.pallas_call(
        body,
        out_shape=jax.ShapeDtypeStruct((M, D), jnp.float32),
        in_specs=[pl.BlockSpec(memory_space=pltpu.VMEM)] * 8,
        out_specs=pl.BlockSpec(memory_space=pltpu.VMEM),
        scratch_shapes=[
            pltpu.VMEM((N_DEV, D, 3 * d_loc), jnp.bfloat16),
            pltpu.VMEM((N_DEV, d_loc, D), jnp.bfloat16),
            pltpu.VMEM((M, d_loc), jnp.bfloat16),
            pltpu.VMEM((M, D), jnp.bfloat16),
            pltpu.SemaphoreType.DMA((2 * N_DEV,)),
            pltpu.SemaphoreType.DMA((2 * N_DEV,)),
            pltpu.SemaphoreType.DMA((2 * N_DEV,)),
        ],
        compiler_params=pltpu.CompilerParams(collective_id=0),
    )(x2, Wq, Wk, Wv, Wo,
      jnp.asarray(cos_bc), jnp.asarray(sin_bc),
      jnp.asarray(R_full).astype(jnp.bfloat16))
    return out2.reshape(B, Sq, D)


# device time: 85636 ns/iter; 3.1186x vs baseline; 1.1025x over previous
import numpy as np

import jax
import jax.numpy as jnp
from jax import lax
from jax.experimental import pallas as pl
from jax.experimental.pallas import tpu as pltpu

N_DEV = 8
DH = 64


def _rope_tables(B, Sq, H_loc):
    inv = 1.0 / (10000.0 ** (np.arange(0, DH, 2) / DH))
    pos = np.arange(Sq)[:, None] * inv[None, :]
    cos = np.repeat(np.cos(pos), 2, axis=-1).astype(np.float32)
    sin = np.repeat(np.sin(pos), 2, axis=-1).astype(np.float32)
    cos_bc = np.tile(cos, (B, H_loc))
    sin_bc = np.tile(sin, (B, H_loc))
    R = np.zeros((DH, DH), np.float32)
    for k in range(DH // 2):
        R[2 * k, 2 * k + 1] = 1.0
        R[2 * k + 1, 2 * k] = -1.0
    R_full = np.kron(np.eye(H_loc, dtype=np.float32), R)
    return cos_bc, sin_bc, R_full


def kernel(x, Wq, Wk, Wv, Wo):
    B, Sq, D = x.shape
    d_loc = Wq.shape[1]
    H_loc = d_loc // DH
    M = B * Sq
    QH = 3 * d_loc // 2
    OH = d_loc // 2

    cos_bc, sin_bc, R_full = _rope_tables(B, Sq, H_loc)
    x2 = x.reshape(M, D)

    def body(x_ref, wq_ref, wk_ref, wv_ref, wo_ref, cos_ref, sin_ref, rot_ref,
             out_ref, qkv_buf, o_buf, ctx_ref, xb_ref,
             sendR, sendL, recv_sems):
        my = lax.axis_index("i")
        left = (my + N_DEV - 1) % N_DEV
        right = (my + 1) % N_DEV

        barrier = pltpu.get_barrier_semaphore()
        pl.semaphore_signal(barrier, inc=1, device_id=(left,),
                            device_id_type=pl.DeviceIdType.MESH)
        pl.semaphore_signal(barrier, inc=1, device_id=(right,),
                            device_id_type=pl.DeviceIdType.MESH)
        xb_ref[...] = x_ref[...].astype(jnp.bfloat16)
        qkv_buf[0, :, 0:d_loc] = wq_ref[...].astype(jnp.bfloat16)
        qkv_buf[0, :, d_loc:2 * d_loc] = wk_ref[...].astype(jnp.bfloat16)
        qkv_buf[0, :, 2 * d_loc:3 * d_loc] = wv_ref[...].astype(jnp.bfloat16)
        o_buf[0] = wo_ref[...].astype(jnp.bfloat16)
        pl.semaphore_wait(barrier, 2)

        def compute(s):
            qkv = jnp.dot(xb_ref[...], qkv_buf[s],
                          preferred_element_type=jnp.float32)
            q = qkv[:, 0:d_loc]
            k = qkv[:, d_loc:2 * d_loc]
            v = qkv[:, 2 * d_loc:3 * d_loc].astype(jnp.bfloat16)
            cos = cos_ref[...]
            sin = sin_ref[...]
            rot = rot_ref[...]
            q = (q * cos + jnp.dot(q.astype(jnp.bfloat16), rot,
                                   preferred_element_type=jnp.float32) * sin
                 ).astype(jnp.bfloat16)
            k = (k * cos + jnp.dot(k.astype(jnp.bfloat16), rot,
                                   preferred_element_type=jnp.float32) * sin
                 ).astype(jnp.bfloat16)
            for b in range(B):
                for hh in range(H_loc):
                    qs = q[b * Sq:(b + 1) * Sq, hh * DH:(hh + 1) * DH]
                    ks = k[b * Sq:(b + 1) * Sq, hh * DH:(hh + 1) * DH]
                    vs = v[b * Sq:(b + 1) * Sq, hh * DH:(hh + 1) * DH]
                    sc = lax.dot_general(
                        qs, ks, (((1,), (1,)), ((), ())),
                        preferred_element_type=jnp.float32) * 0.125
                    m = jnp.max(sc, axis=1, keepdims=True)
                    w = jnp.exp(sc - m)
                    w = (w / jnp.sum(w, axis=1, keepdims=True)
                         ).astype(jnp.bfloat16)
                    ctx_ref[b * Sq:(b + 1) * Sq, hh * DH:(hh + 1) * DH] = (
                        jnp.dot(w, vs, preferred_element_type=jnp.float32)
                        .astype(jnp.bfloat16))
            return jnp.dot(ctx_ref[...], o_buf[s],
                           preferred_element_type=jnp.float32)

        def make_rdma(src_slot, dst_slot, send_sem, target):
            qkv_rdma = pltpu.make_async_remote_copy(
                src_ref=qkv_buf.at[src_slot], dst_ref=qkv_buf.at[dst_slot],
                send_sem=send_sem.at[2 * dst_slot],
                recv_sem=recv_sems.at[2 * dst_slot],
                device_id=(target,), device_id_type=pl.DeviceIdType.MESH)
            o_rdma = pltpu.make_async_remote_copy(
                src_ref=o_buf.at[src_slot], dst_ref=o_buf.at[dst_slot],
                send_sem=send_sem.at[2 * dst_slot + 1],
                recv_sem=recv_sems.at[2 * dst_slot + 1],
                device_id=(target,), device_id_type=pl.DeviceIdType.MESH)
            return [qkv_rdma, o_rdma]

        def make_half_rdma(src_slot, lo, target, send_sem, sem_base):
            qs = slice(0, QH) if lo else slice(QH, 3 * d_loc)
            os_ = slice(0, OH) if lo else slice(OH, d_loc)
            qkv_rdma = pltpu.make_async_remote_copy(
                src_ref=qkv_buf.at[src_slot, slice(None), qs],
                dst_ref=qkv_buf.at[4, slice(None), qs],
                send_sem=send_sem.at[sem_base],
                recv_sem=recv_sems.at[sem_base],
                device_id=(target,), device_id_type=pl.DeviceIdType.MESH)
            o_rdma = pltpu.make_async_remote_copy(
                src_ref=o_buf.at[src_slot, os_],
                dst_ref=o_buf.at[4, os_],
                send_sem=send_sem.at[sem_base + 1],
                recv_sem=recv_sems.at[sem_base + 1],
                device_id=(target,), device_id_type=pl.DeviceIdType.MESH)
            return [qkv_rdma, o_rdma]

        acc_init = [False]

        def accumulate(c):
            if acc_init[0]:
                out_ref[...] = out_ref[...] + c
            else:
                out_ref[...] = c
                acc_init[0] = True

        started = []
        for r in range(1, 5):
            if r <= 3:
                rdmas = make_rdma(r - 1, r, sendR, right)
                rdmas += make_rdma(0 if r == 1 else 4 + r - 1, 4 + r,
                                   sendL, left)
            else:
                rdmas = make_half_rdma(3, True, right, sendR, 8)
                rdmas += make_half_rdma(7, False, left, sendL, 0)
            for d in rdmas:
                d.start()
            if r == 1:
                accumulate(compute(0))
            else:
                accumulate(compute(r - 1))
                accumulate(compute(4 + r - 1))
            for d in rdmas:
                d.wait_recv()
            started += rdmas
        accumulate(compute(4))
        for d in started:
            d.wait_send()

    out2 = pl.pallas_call(
        body,
        out_shape=jax.ShapeDtypeStruct((M, D), jnp.float32),
        in_specs=[pl.BlockSpec(memory_space=pltpu.VMEM)] * 8,
        out_specs=pl.BlockSpec(memory_space=pltpu.VMEM),
        scratch_shapes=[
            pltpu.VMEM((N_DEV, D, 3 * d_loc), jnp.bfloat16),
            pltpu.VMEM((N_DEV, d_loc, D), jnp.bfloat16),
            pltpu.VMEM((M, d_loc), jnp.bfloat16),
            pltpu.VMEM((M, D), jnp.bfloat16),
            pltpu.SemaphoreType.DMA((2 * N_DEV,)),
            pltpu.SemaphoreType.DMA((2 * N_DEV,)),
            pltpu.SemaphoreType.DMA((2 * N_DEV,)),
        ],
        compiler_params=pltpu.CompilerParams(collective_id=0),
    )(x2, Wq, Wk, Wv, Wo,
      jnp.asarray(cos_bc), jnp.asarray(sin_bc),
      jnp.asarray(R_full).astype(jnp.bfloat16))
    return out2.reshape(B, Sq, D)
